# baseline (speedup 1.0000x reference)
import functools
import math

import jax
import jax.numpy as jnp
from jax import lax
from jax.experimental import pallas as pl
from jax.experimental.pallas import tpu as pltpu
from jax.experimental.pallas import tpu_sc as plsc

VOCAB = 1000000
EMB = 64
B = 4096
L = 200
N = B * L
SCALE = math.sqrt(EMB)

_info = plsc.get_sparse_core_info()
NC, NS, LANES = _info.num_cores, _info.num_subcores, _info.num_lanes
NW = NC * NS
PER_W = N // NW
CHUNK = 128
STEPS = PER_W // CHUNK


def _body(tok_hbm, table_hbm, out_hbm, idx_v, rows_v, sem):
    wid = lax.axis_index("s") * NC + lax.axis_index("c")
    w_base = wid * PER_W

    def step(g, carry):
        base = w_base + g * CHUNK
        pltpu.sync_copy(tok_hbm.at[pl.ds(base, CHUNK)], idx_v)
        pltpu.async_copy(table_hbm.at[idx_v], rows_v, sem).wait()

        def scale_row(r, c):
            for j in range(EMB // LANES):
                sl = pl.ds(j * LANES, LANES)
                rows_v[r, sl] = rows_v[r, sl] * SCALE
            return c

        lax.fori_loop(0, CHUNK, scale_row, 0)
        pltpu.sync_copy(rows_v, out_hbm.at[pl.ds(base, CHUNK)])
        return carry

    lax.fori_loop(0, STEPS, step, 0)


@functools.partial(jax.jit, static_argnames=())
def kernel(tokens, table):
    tok_flat = tokens.reshape(N).astype(jnp.int32)
    mesh = plsc.VectorSubcoreMesh(core_axis_name="c", subcore_axis_name="s")
    run = pl.kernel(
        _body,
        out_type=jax.ShapeDtypeStruct((N, EMB), jnp.float32),
        mesh=mesh,
        scratch_types=[
            pltpu.VMEM((CHUNK,), jnp.int32),
            pltpu.VMEM((CHUNK, EMB), jnp.float32),
            pltpu.SemaphoreType.DMA,
        ],
        compiler_params=pltpu.CompilerParams(use_tc_tiling_on_sc=False),
    )
    out = run(tok_flat, table)
    return out.reshape(B, L, EMB)


# speedup vs baseline: 1.2626x; 1.2626x over previous
import functools
import math

import jax
import jax.numpy as jnp
from jax import lax
from jax.experimental import pallas as pl
from jax.experimental.pallas import tpu as pltpu
from jax.experimental.pallas import tpu_sc as plsc

VOCAB = 1000000
EMB = 64
B = 4096
L = 200
N = B * L
SCALE = math.sqrt(EMB)

_info = plsc.get_sparse_core_info()
NC, NS, LANES = _info.num_cores, _info.num_subcores, _info.num_lanes
NW = NC * NS
PER_W = N // NW
CHUNK = 128
STEPS = PER_W // CHUNK
NBUF = 4
GROUPS = STEPS // NBUF
RU = 8


def _body(tok_hbm, table_hbm, out_hbm, idx_v, rows_v, gsems, wsems):
    wid = lax.axis_index("s") * NC + lax.axis_index("c")
    w_base = wid * PER_W

    def gather_start(g, b):
        pltpu.async_copy(table_hbm.at[idx_v.at[g]], rows_v.at[b], gsems[b])

    def gather_wait(g, b):
        pltpu.make_async_copy(table_hbm.at[idx_v.at[g]], rows_v.at[b], gsems[b]).wait()

    def wb_start(g, b):
        base = w_base + g * CHUNK
        pltpu.async_copy(rows_v.at[b], out_hbm.at[pl.ds(base, CHUNK)], wsems[b])

    def wb_wait(g, b):
        base = w_base + g * CHUNK
        pltpu.make_async_copy(rows_v.at[b], out_hbm.at[pl.ds(base, CHUNK)], wsems[b]).wait()

    def scale(b):
        def srow(r0, c):
            for r in range(RU):
                for j in range(EMB // LANES):
                    sl = pl.ds(j * LANES, LANES)
                    rows_v[b, r0 * RU + r, sl] = rows_v[b, r0 * RU + r, sl] * SCALE
            return c

        lax.fori_loop(0, CHUNK // RU, srow, 0)

    pltpu.sync_copy(tok_hbm.at[wid], idx_v)
    for b in range(NBUF):
        gather_start(b, b)

    def group(go, carry):
        for b in range(NBUF):
            g = go * NBUF + b
            gather_wait(g, b)
            scale(b)
            wb_start(g, b)
        for b in range(NBUF):
            g = go * NBUF + b
            wb_wait(g, b)
            gather_start(g + NBUF, b)
        return carry

    lax.fori_loop(0, GROUPS - 1, group, 0)

    go = GROUPS - 1
    for b in range(NBUF):
        g = go * NBUF + b
        gather_wait(g, b)
        scale(b)
        wb_start(g, b)
    for b in range(NBUF):
        g = go * NBUF + b
        wb_wait(g, b)


@functools.partial(jax.jit, static_argnames=())
def kernel(tokens, table):
    tok3 = tokens.reshape(NW, STEPS, CHUNK).astype(jnp.int32)
    mesh = plsc.VectorSubcoreMesh(core_axis_name="c", subcore_axis_name="s")
    run = pl.kernel(
        _body,
        out_type=jax.ShapeDtypeStruct((N, EMB), jnp.float32),
        mesh=mesh,
        scratch_types=[
            pltpu.VMEM((STEPS, CHUNK), jnp.int32),
            pltpu.VMEM((NBUF, CHUNK, EMB), jnp.float32),
            [pltpu.SemaphoreType.DMA] * NBUF,
            [pltpu.SemaphoreType.DMA] * NBUF,
        ],
        compiler_params=pltpu.CompilerParams(use_tc_tiling_on_sc=False),
    )
    out = run(tok3, table)
    return out.reshape(B, L, EMB)


# speedup vs baseline: 1.2726x; 1.0080x over previous
import functools
import math

import jax
import jax.numpy as jnp
from jax import lax
from jax.experimental import pallas as pl
from jax.experimental.pallas import tpu as pltpu
from jax.experimental.pallas import tpu_sc as plsc

VOCAB = 1000000
EMB = 64
B = 4096
L = 200
N = B * L
SCALE = math.sqrt(EMB)

_info = plsc.get_sparse_core_info()
NC, NS, LANES = _info.num_cores, _info.num_subcores, _info.num_lanes
NW = NC * NS
PER_W = N // NW
CHUNK = 128
STEPS = PER_W // CHUNK
NB = 5
GROUPS = STEPS // NB
RU = 8


def _body(tok_hbm, table_hbm, out_hbm, idx_v, gbuf, wbuf, gsems, wsems):
    wid = lax.axis_index("s") * NC + lax.axis_index("c")
    w_base = wid * PER_W

    def gather_start(g, b):
        pltpu.async_copy(table_hbm.at[idx_v.at[g]], gbuf.at[b], gsems[b])

    def gather_wait(g, b):
        pltpu.make_async_copy(table_hbm.at[idx_v.at[g]], gbuf.at[b], gsems[b]).wait()

    def wb_start(g, b):
        base = w_base + g * CHUNK
        pltpu.async_copy(wbuf.at[b], out_hbm.at[pl.ds(base, CHUNK)], wsems[b])

    def wb_wait(g, b):
        base = w_base + g * CHUNK
        pltpu.make_async_copy(wbuf.at[b], out_hbm.at[pl.ds(base, CHUNK)], wsems[b]).wait()

    def scale(b):
        def srow(r0, c):
            for r in range(RU):
                row = r0 * RU + r
                for j in range(EMB // LANES):
                    sl = pl.ds(j * LANES, LANES)
                    wbuf[b, row, sl] = gbuf[b, row, sl] * SCALE
            return c

        lax.fori_loop(0, CHUNK // RU, srow, 0)

    pltpu.sync_copy(tok_hbm.at[wid], idx_v)
    for b in range(NB):
        gather_start(b, b)

    for b in range(NB):
        gather_wait(b, b)
        scale(b)
        gather_start(b + NB, b)
        wb_start(b, b)

    def group(go, carry):
        for b in range(NB):
            g = go * NB + b
            gather_wait(g, b)
            wb_wait(g - NB, b)
            scale(b)
            gather_start(g + NB, b)
            wb_start(g, b)
        return carry

    lax.fori_loop(1, GROUPS - 1, group, 0)

    go = GROUPS - 1
    for b in range(NB):
        g = go * NB + b
        gather_wait(g, b)
        wb_wait(g - NB, b)
        scale(b)
        wb_start(g, b)
    for b in range(NB):
        wb_wait(go * NB + b, b)


@functools.partial(jax.jit, static_argnames=())
def kernel(tokens, table):
    tok3 = tokens.reshape(NW, STEPS, CHUNK).astype(jnp.int32)
    mesh = plsc.VectorSubcoreMesh(core_axis_name="c", subcore_axis_name="s")
    run = pl.kernel(
        _body,
        out_type=jax.ShapeDtypeStruct((N, EMB), jnp.float32),
        mesh=mesh,
        scratch_types=[
            pltpu.VMEM((STEPS, CHUNK), jnp.int32),
            pltpu.VMEM((NB, CHUNK, EMB), jnp.float32),
            pltpu.VMEM((NB, CHUNK, EMB), jnp.float32),
            [pltpu.SemaphoreType.DMA] * NB,
            [pltpu.SemaphoreType.DMA] * NB,
        ],
        compiler_params=pltpu.CompilerParams(use_tc_tiling_on_sc=False),
    )
    out = run(tok3, table)
    return out.reshape(B, L, EMB)
